# Initial kernel scaffold; baseline (speedup 1.0000x reference)
import jax
import jax.numpy as jnp
from jax.experimental import pallas as pl


def kernel(x, target):
    raise NotImplementedError("write your pallas kernel here")


# speedup vs baseline: 21.9574x; 21.9574x over previous
import jax
import jax.numpy as jnp
from jax.experimental import pallas as pl
from jax.experimental.pallas import tpu as pltpu

_BINS = 10
_SCALE = 10.0 - 0.0001
_ROWS = 16384
_COLS = 1024
_BLOCK_ROWS = 1024
_N_STEPS = _ROWS // _BLOCK_ROWS


def _ghm_kernel(x_ref, t_ref, out_ref, cnt_ref, sum_ref):
    step = pl.program_id(0)

    @pl.when(step == 0)
    def _init():
        cnt_ref[...] = jnp.zeros_like(cnt_ref)
        sum_ref[...] = jnp.zeros_like(sum_ref)

    x = x_ref[...]
    t = t_ref[...]

    sig = jax.nn.sigmoid(x)
    g = jnp.abs(sig - t)
    binf = jnp.floor(g * _SCALE)
    bce = jnp.maximum(x, 0.0) - x * t + jnp.log1p(jnp.exp(-jnp.abs(x)))

    def fold(a):
        return a.reshape(_BLOCK_ROWS // 8, 8, _COLS).sum(axis=0)

    for b in range(_BINS):
        mask = binf == jnp.float32(b)
        cnt_ref[8 * b:8 * (b + 1), :] += fold(
            jnp.where(mask, jnp.float32(1.0), jnp.float32(0.0)))
        sum_ref[8 * b:8 * (b + 1), :] += fold(
            jnp.where(mask, bce, jnp.float32(0.0)))

    @pl.when(step == _N_STEPS - 1)
    def _finish():
        counts = [jnp.sum(cnt_ref[8 * b:8 * (b + 1), :]) for b in range(_BINS)]
        sums = [jnp.sum(sum_ref[8 * b:8 * (b + 1), :]) for b in range(_BINS)]
        nonempty = jnp.float32(0.0)
        for b in range(_BINS):
            nonempty += jnp.where(counts[b] > 0.0, jnp.float32(1.0),
                                  jnp.float32(0.0))
        loss = jnp.float32(0.0)
        for b in range(_BINS):
            gd = jnp.maximum(counts[b] * nonempty, jnp.float32(0.0001))
            loss += sums[b] / gd
        out_ref[...] = jnp.full((8, 128), loss, dtype=jnp.float32)


@jax.jit
def kernel(x, target):
    out = pl.pallas_call(
        _ghm_kernel,
        grid=(_N_STEPS,),
        in_specs=[
            pl.BlockSpec((_BLOCK_ROWS, _COLS), lambda i: (i, 0)),
            pl.BlockSpec((_BLOCK_ROWS, _COLS), lambda i: (i, 0)),
        ],
        out_specs=pl.BlockSpec((8, 128), lambda i: (0, 0)),
        out_shape=jax.ShapeDtypeStruct((8, 128), jnp.float32),
        scratch_shapes=[
            pltpu.VMEM((8 * _BINS, _COLS), jnp.float32),
            pltpu.VMEM((8 * _BINS, _COLS), jnp.float32),
        ],
    )(x, target)
    return out[0, 0]
